# Initial kernel scaffold; baseline (speedup 1.0000x reference)
import functools

import jax
import jax.numpy as jnp
from jax import lax
from jax.experimental import pallas as pl
from jax.experimental.pallas import tpu as pltpu, tpu_sc as plsc

N = 500000
DIM = 128
VOCAB = 101
EPAD = 128
CHUNK = 128
LANES = 16

NUM_FULL = N // CHUNK
TAIL = N - NUM_FULL * CHUNK


def _search16(edges_v, v):
    pos = jnp.zeros((LANES,), jnp.int32)
    step = EPAD // 2
    while step >= 1:
        probe = pos + (step - 1)
        ev = plsc.load_gather(edges_v, [probe])
        pos = jnp.where(ev < v, pos + step, pos)
        step //= 2
    return pos


def _body(time_hbm, edges_hbm, table_hbm, out_hbm,
          edges_v, tv, idx, rows, tvt, idxt, rowst, sem):
    nc = lax.axis_index("c")
    ns = lax.axis_index("s")
    wid = ns * 2 + nc
    nw = 32

    pltpu.sync_copy(edges_hbm, edges_v)

    base_chunks = NUM_FULL // nw
    extra = NUM_FULL - base_chunks * nw
    nch = jnp.where(wid < extra, base_chunks + 1, base_chunks)

    def chunk_step(t, carry):
        cid = wid + t * nw
        base = cid * CHUNK
        pltpu.sync_copy(time_hbm.at[pl.ds(base, CHUNK)], tv)
        for k in range(CHUNK // LANES):
            v = tv[pl.ds(k * LANES, LANES)]
            idx[pl.ds(k * LANES, LANES)] = _search16(edges_v, v)
        pltpu.async_copy(table_hbm.at[idx], rows, sem).wait()
        pltpu.sync_copy(rows, out_hbm.at[pl.ds(base, CHUNK)])
        return carry

    lax.fori_loop(0, nch, chunk_step, 0)

    @pl.when(wid == nw - 1)
    def _tail():
        base = NUM_FULL * CHUNK
        pltpu.sync_copy(time_hbm.at[pl.ds(base, TAIL)], tvt)
        for k in range(TAIL // LANES):
            v = tvt[pl.ds(k * LANES, LANES)]
            idxt[pl.ds(k * LANES, LANES)] = _search16(edges_v, v)
        pltpu.async_copy(table_hbm.at[idxt], rowst, sem).wait()
        pltpu.sync_copy(rowst, out_hbm.at[pl.ds(base, TAIL)])


@jax.jit
def _run(time_value, edges_pad, table):
    mesh = plsc.VectorSubcoreMesh(core_axis_name="c", subcore_axis_name="s")
    return pl.kernel(
        _body,
        out_type=jax.ShapeDtypeStruct((N, DIM), jnp.float32),
        mesh=mesh,
        scratch_types=[
            pltpu.VMEM((EPAD,), jnp.float32),
            pltpu.VMEM((CHUNK,), jnp.float32),
            pltpu.VMEM((CHUNK,), jnp.int32),
            pltpu.VMEM((CHUNK, DIM), jnp.float32),
            pltpu.VMEM((TAIL,), jnp.float32),
            pltpu.VMEM((TAIL,), jnp.int32),
            pltpu.VMEM((TAIL, DIM), jnp.float32),
            pltpu.SemaphoreType.DMA,
        ],
    )(time_value, edges_pad, table)


def kernel(time_value, absolute_bin_edges, ab_duration_embed):
    edges_pad = jnp.concatenate(
        [absolute_bin_edges.astype(jnp.float32),
         jnp.full((EPAD - absolute_bin_edges.shape[0],), jnp.inf, jnp.float32)]
    )
    return _run(time_value, edges_pad, ab_duration_embed)


# speedup vs baseline: 36.0898x; 36.0898x over previous
import functools

import jax
import jax.numpy as jnp
from jax import lax
from jax.experimental import pallas as pl
from jax.experimental.pallas import tpu as pltpu, tpu_sc as plsc

N = 500000
DIM = 128
VOCAB = 101
EPAD = 128
CHUNK = 128
LANES = 16

NUM_FULL = N // CHUNK
TAIL = N - NUM_FULL * CHUNK


def _search16(edges_v, v):
    pos = jnp.zeros((LANES,), jnp.int32)
    step = EPAD // 2
    while step >= 1:
        probe = pos + (step - 1)
        ev = plsc.load_gather(edges_v, [probe])
        pos = jnp.where(ev < v, pos + step, pos)
        step //= 2
    return pos


def _body(time_hbm, edges_hbm, table_hbm, out_hbm,
          edges_v, tv, idx, rows, tvt, idxt, rowst, sem):
    nc = lax.axis_index("c")
    ns = lax.axis_index("s")
    wid = ns * 2 + nc
    nw = 32

    pltpu.sync_copy(edges_hbm, edges_v)

    base_chunks = NUM_FULL // nw
    extra = NUM_FULL - base_chunks * nw
    nch = jnp.where(wid < extra, base_chunks + 1, base_chunks)

    def chunk_step(t, carry):
        cid = wid + t * nw
        base = cid * CHUNK
        pltpu.sync_copy(time_hbm.at[pl.ds(base, CHUNK)], tv)
        for k in range(CHUNK // LANES):
            v = tv[pl.ds(k * LANES, LANES)]
            idx[pl.ds(k * LANES, LANES)] = _search16(edges_v, v)
        pltpu.async_copy(table_hbm.at[idx], rows, sem).wait()
        pltpu.sync_copy(rows, out_hbm.at[pl.ds(base, CHUNK)])
        return carry

    lax.fori_loop(0, nch, chunk_step, 0)

    @pl.when(wid == nw - 1)
    def _tail():
        base = NUM_FULL * CHUNK
        pltpu.sync_copy(time_hbm.at[pl.ds(base, TAIL)], tvt)
        for k in range(TAIL // LANES):
            v = tvt[pl.ds(k * LANES, LANES)]
            idxt[pl.ds(k * LANES, LANES)] = _search16(edges_v, v)
        pltpu.async_copy(table_hbm.at[idxt], rowst, sem).wait()
        pltpu.sync_copy(rowst, out_hbm.at[pl.ds(base, TAIL)])


@jax.jit
def _run(time_value, edges_pad, table):
    mesh = plsc.VectorSubcoreMesh(core_axis_name="c", subcore_axis_name="s")
    return pl.kernel(
        _body,
        out_type=jax.ShapeDtypeStruct((N, DIM), jnp.float32),
        mesh=mesh,
        compiler_params=pltpu.CompilerParams(needs_layout_passes=False),
        scratch_types=[
            pltpu.VMEM((EPAD,), jnp.float32),
            pltpu.VMEM((CHUNK,), jnp.float32),
            pltpu.VMEM((CHUNK,), jnp.int32),
            pltpu.VMEM((CHUNK, DIM), jnp.float32),
            pltpu.VMEM((TAIL,), jnp.float32),
            pltpu.VMEM((TAIL,), jnp.int32),
            pltpu.VMEM((TAIL, DIM), jnp.float32),
            pltpu.SemaphoreType.DMA,
        ],
    )(time_value, edges_pad, table)


def kernel(time_value, absolute_bin_edges, ab_duration_embed):
    edges_pad = jnp.concatenate(
        [absolute_bin_edges.astype(jnp.float32),
         jnp.full((EPAD - absolute_bin_edges.shape[0],), jnp.inf, jnp.float32)]
    )
    return _run(time_value, edges_pad, ab_duration_embed)
